# baseline (speedup 1.0000x reference)
import functools

import jax
import jax.numpy as jnp
from jax import lax
from jax.experimental import pallas as pl
from jax.experimental.pallas import tpu as pltpu
from jax.experimental.pallas import tpu_sc as plsc

N_NODES = 10000
D = 256
T = 8

NW = 32
NPW = 320
NPAD = NW * NPW
S = 1024
CH = 64
NCH = S // CH
L = 16
VR = D // L
BM = 512

_MESH = dict(core_axis_name="c", subcore_axis_name="s")


def _wid():
    return lax.axis_index("s") * 2 + lax.axis_index("c")


def _bcast(s):
    return lax.broadcast_in_dim(s, (L,), ())


def _deg_body(ws_hbm, dstl_hbm, bounds_hbm, dinv_hbm,
              wbuf, dbuf, degbuf, bbuf, dinvbuf):
    wid = _wid()
    pltpu.sync_copy(bounds_hbm, bbuf)
    bv = bbuf[pl.ds(wid, L)]
    e0 = bv[0]
    e1 = bv[1]
    def zb(i, carry):
        degbuf[pl.ds(i * L, L)] = jnp.zeros((L,), jnp.float32)
        return carry
    lax.fori_loop(0, (NPW + L) // L, zb, None)

    cb0 = (e0 // S) * S
    nsc = (e1 - cb0 + S - 1) // S

    def sc_body(m, carry):
        base = cb0 + m * S
        pltpu.sync_copy(ws_hbm.at[pl.ds(base, S)], wbuf.at[pl.ds(0, S)])
        pltpu.sync_copy(dstl_hbm.at[pl.ds(base, S)], dbuf.at[pl.ds(0, S)])
        lo = jnp.maximum(e0, base) - base
        hi = jnp.minimum(e1, base + S) - base

        def e_body(j, c2):
            d = dbuf[pl.ds(j, L)][0]
            io = lax.broadcasted_iota(jnp.int32, (L,), 0)
            wv = jnp.where(io == 0, wbuf[pl.ds(j, L)], 0.0)
            degbuf[pl.ds(d, L)] = degbuf[pl.ds(d, L)] + wv
            return c2
        lax.fori_loop(lo, hi, e_body, None)
        return carry
    lax.fori_loop(0, nsc, sc_body, None)

    for v in range(NPW // L):
        sl = pl.ds(v * L, L)
        deg = degbuf[sl]
        ib = lax.bitcast_convert_type(deg, jnp.int32)
        y = lax.bitcast_convert_type(
            jnp.int32(0x5F3759DF) - lax.shift_right_logical(ib, 1),
            jnp.float32)
        for _ in range(4):
            y = y * (1.5 - 0.5 * deg * y * y)
        dinvbuf[sl] = y

    n0 = wid * NPW
    pltpu.sync_copy(dinvbuf, dinv_hbm.at[pl.ds(n0, NPW)])


def _deg_kernel(ws, dstl, bounds):
    f = pl.kernel(
        _deg_body,
        out_type=jax.ShapeDtypeStruct((NPAD,), jnp.float32),
        mesh=plsc.VectorSubcoreMesh(**_MESH),
        compiler_params=pltpu.CompilerParams(needs_layout_passes=False),
        scratch_types=[
            pltpu.VMEM((S + L,), jnp.float32),
            pltpu.VMEM((S + L,), jnp.int32),
            pltpu.VMEM((NPW + L,), jnp.float32),
            pltpu.VMEM((48,), jnp.int32),
            pltpu.VMEM((NPW,), jnp.float32),
        ],
    )
    return f(ws, dstl, bounds)


def _norm_body(dinv_hbm, src_hbm, dst_hbm, ws_hbm, norm_hbm,
               tab, sbuf, dbuf, wbuf, nbuf, epw):
    wid = _wid()
    pltpu.sync_copy(dinv_hbm, tab)
    for m in range(epw // S):
        base = wid * epw + m * S
        pltpu.sync_copy(src_hbm.at[pl.ds(base, S)], sbuf)
        pltpu.sync_copy(dst_hbm.at[pl.ds(base, S)], dbuf)
        pltpu.sync_copy(ws_hbm.at[pl.ds(base, S)], wbuf)

        def g_body(u, carry):
            off = pl.ds(u * L, L)
            gs = plsc.load_gather(tab, [sbuf[off]])
            gd = plsc.load_gather(tab, [dbuf[off]])
            nbuf[off] = gs * wbuf[off] * gd
            return carry
        lax.fori_loop(0, S // L, g_body, None)
        pltpu.sync_copy(nbuf, norm_hbm.at[pl.ds(base, S)])


def _norm_kernel(dinv, src_s, dst_s, ws, epad):
    epw = epad // NW
    f = pl.kernel(
        functools.partial(_norm_body, epw=epw),
        out_type=jax.ShapeDtypeStruct((epad,), jnp.float32),
        mesh=plsc.VectorSubcoreMesh(**_MESH),
        compiler_params=pltpu.CompilerParams(needs_layout_passes=False),
        scratch_types=[
            pltpu.VMEM((NPAD,), jnp.float32),
            pltpu.VMEM((S,), jnp.int32),
            pltpu.VMEM((S,), jnp.int32),
            pltpu.VMEM((S,), jnp.float32),
            pltpu.VMEM((S,), jnp.float32),
        ],
    )
    return f(dinv, src_s, dst_s, ws)


def _agg_body(xw_hbm, src_hbm, dstl_hbm, norm_hbm, bounds_hbm, b_hbm,
              out_hbm, acc, gbuf, sbuf, mdbuf, mnbuf, ibuf, bbuf,
              bvbuf, sem0, sem1, relu):
    wid = _wid()
    n0 = wid * NPW
    pltpu.sync_copy(bounds_hbm, bbuf)
    bv = bbuf[pl.ds(wid, L)]
    e0 = bv[0]
    e1 = bv[1]
    pltpu.sync_copy(b_hbm, bvbuf)
    cb0 = (e0 // S) * S
    nsc = (e1 - cb0 + S - 1) // S
    sems = [sem0, sem1]

    def t_body(t, carry):
        def zb(i, c2):
            for v in range(VR):
                sl = pl.ds(v * L, L)
                acc[pl.ds(i * D + v * L, L)] = bvbuf[sl]
            return c2
        lax.fori_loop(0, NPW, zb, None)

        tN = t * NPAD

        def sc_body(m, c2):
            base = cb0 + m * S
            pltpu.sync_copy(src_hbm.at[pl.ds(base, S)], sbuf)
            pltpu.sync_copy(dstl_hbm.at[pl.ds(base, S)],
                            mdbuf.at[pl.ds(0, S)])
            pltpu.sync_copy(norm_hbm.at[pl.ds(base, S)],
                            mnbuf.at[pl.ds(0, S)])

            def ib_body(u, c3):
                off = pl.ds(u * L, L)
                ibuf[off] = sbuf[off] + tN
                gi = base + u * L + lax.broadcasted_iota(jnp.int32, (L,), 0)
                ok = (gi >= e0) & (gi < e1)
                mnbuf[off] = jnp.where(ok, mnbuf[off], 0.0)
                return c3
            lax.fori_loop(0, S // L, ib_body, None)

            pltpu.async_copy(
                xw_hbm.at[ibuf.at[pl.ds(0, CH)]], gbuf.at[0], sems[0])
            pltpu.async_copy(
                xw_hbm.at[ibuf.at[pl.ds(CH, CH)]], gbuf.at[1], sems[1])

            def ch_body(k2, c3):
                for b in range(2):
                    cidx = 2 * k2 + b
                    pltpu.make_async_copy(
                        xw_hbm.at[ibuf.at[pl.ds(b * CH, CH)]], gbuf.at[b],
                        sems[b]).wait()

                    def e_body(j, c4, b=b):
                        jj = cidx * CH + j
                        nb = _bcast(mnbuf[pl.ds(jj, L)][0])
                        dd = mdbuf[pl.ds(jj, L)][0]
                        for v in range(VR):
                            sl = pl.ds(v * L, L)
                            plsc.addupdate(
                                acc.at[pl.ds(dd * D + v * L, L)],
                                nb * gbuf[b, j, sl])
                        return c4
                    lax.fori_loop(0, CH, e_body, None)

                    @pl.when(k2 < NCH // 2 - 1)
                    def _issue(b=b, cidx=cidx):
                        pltpu.async_copy(
                            xw_hbm.at[ibuf.at[pl.ds((cidx + 2) * CH, CH)]],
                            gbuf.at[b], sems[b])
                return c3
            lax.fori_loop(0, NCH // 2, ch_body, None)
            return c2
        lax.fori_loop(0, nsc, sc_body, None)

        if relu:
            def rl(i, c2):
                sl = pl.ds(i * L, L)
                acc[sl] = jnp.maximum(acc[sl], 0.0)
                return c2
            lax.fori_loop(0, NPW * VR, rl, None)

        pltpu.sync_copy(acc, out_hbm.at[pl.ds((tN + n0) * D, NPW * D)])
        return carry
    lax.fori_loop(0, T, t_body, None)


def _agg_kernel(xw, src_s, dstl_s, norm_s, bounds, b, relu):
    f = pl.kernel(
        functools.partial(_agg_body, relu=relu),
        out_type=jax.ShapeDtypeStruct((T * NPAD * D,), jnp.float32),
        mesh=plsc.VectorSubcoreMesh(**_MESH),
        compiler_params=pltpu.CompilerParams(needs_layout_passes=False),
        scratch_types=[
            pltpu.VMEM((NPW * D,), jnp.float32),
            pltpu.VMEM((2, CH, D), jnp.float32),
            pltpu.VMEM((S,), jnp.int32),
            pltpu.VMEM((S + L,), jnp.int32),
            pltpu.VMEM((S + L,), jnp.float32),
            pltpu.VMEM((S,), jnp.int32),
            pltpu.VMEM((48,), jnp.int32),
            pltpu.VMEM((D,), jnp.float32),
            pltpu.SemaphoreType.DMA,
            pltpu.SemaphoreType.DMA,
        ],
    )
    return f(xw, src_s, dstl_s, norm_s, bounds, b)


def _mm_body(x_ref, w_ref, o_ref):
    o_ref[...] = jnp.dot(x_ref[...], w_ref[...],
                         preferred_element_type=jnp.float32)


def _matmul(h, W):
    M = h.shape[0]
    return pl.pallas_call(
        _mm_body,
        grid=(M // BM,),
        in_specs=[pl.BlockSpec((BM, D), lambda i: (i, 0)),
                  pl.BlockSpec((D, D), lambda i: (0, 0))],
        out_specs=pl.BlockSpec((BM, D), lambda i: (i, 0)),
        out_shape=jax.ShapeDtypeStruct((M, D), jnp.float32),
    )(h, W)


def kernel(x, mask, spatial_edge_index, spatial_edge_weight, W1, b1, W2, b2):
    src = spatial_edge_index[0].astype(jnp.int32)
    dst = spatial_edge_index[1].astype(jnp.int32)
    w = spatial_edge_weight.astype(jnp.float32)
    E = src.shape[0]

    loop = jnp.arange(N_NODES, dtype=jnp.int32)
    src2 = jnp.concatenate([src, loop])
    dst2 = jnp.concatenate([dst, loop])
    w2 = jnp.concatenate([w, jnp.ones((N_NODES,), jnp.float32)])
    e2 = E + N_NODES

    p1 = jnp.argsort(dst2)
    dsts = dst2[p1]
    bounds = jnp.searchsorted(
        dsts, jnp.arange(NW + 1, dtype=jnp.int32) * NPW).astype(jnp.int32)
    bounds = jnp.pad(bounds, (0, 48 - (NW + 1)))

    src_f = src2[p1]
    dst_f = dsts
    w_f = w2[p1]
    dstl_f = dst_f % NPW

    epad = NW * S * -(-e2 // (NW * S))
    pad = epad - e2
    src_p = jnp.pad(src_f, (0, pad))
    dst_p = jnp.pad(dst_f, (0, pad))
    dstl_p = jnp.pad(dstl_f, (0, pad))
    w_p = jnp.pad(w_f, (0, pad))

    dinv = _deg_kernel(w_p, dstl_p, bounds)
    norm_p = _norm_kernel(dinv, src_p, dst_p, w_p, epad)

    h = jnp.pad(x, ((0, 0), (0, NPAD - N_NODES), (0, 0))).reshape(T * NPAD, D)
    xw1 = _matmul(h, W1)
    h1 = _agg_kernel(xw1, src_p, dstl_p, norm_p, bounds, b1,
                     relu=True).reshape(T * NPAD, D)
    xw2 = _matmul(h1, W2)
    out = _agg_kernel(xw2, src_p, dstl_p, norm_p, bounds, b2, relu=False)
    return out.reshape(T, NPAD, D)[:, :N_NODES, :]


# speedup vs baseline: 1.2288x; 1.2288x over previous
import functools

import jax
import jax.numpy as jnp
from jax import lax
from jax.experimental import pallas as pl
from jax.experimental.pallas import tpu as pltpu
from jax.experimental.pallas import tpu_sc as plsc

N_NODES = 10000
D = 256
T = 8

NW = 32
NPW = 320
NPAD = NW * NPW
S = 1024
CH = 64
NCH = S // CH
L = 16
VR = D // L
BM = 512

_MESH = dict(core_axis_name="c", subcore_axis_name="s")


def _wid():
    return lax.axis_index("s") * 2 + lax.axis_index("c")


def _bcast(s):
    return lax.broadcast_in_dim(s, (L,), ())


def _deg_body(ws_hbm, dstl_hbm, bounds_hbm, dinv_hbm, sn_hbm,
              wbuf, dbuf, degbuf, bbuf, dinvbuf, snbuf):
    wid = _wid()
    pltpu.sync_copy(bounds_hbm, bbuf)
    bv = bbuf[pl.ds(wid, L)]
    e0 = bv[0]
    e1 = bv[1]

    def zb(i, carry):
        degbuf[pl.ds(i * L, L)] = jnp.zeros((L,), jnp.float32)
        return carry
    lax.fori_loop(0, (NPW + L) // L, zb, None)

    cb0 = (e0 // S) * S
    nsc = (e1 - cb0 + S - 1) // S

    def sc_body(m, carry):
        base = cb0 + m * S
        pltpu.sync_copy(ws_hbm.at[pl.ds(base, S)], wbuf.at[pl.ds(0, S)])
        pltpu.sync_copy(dstl_hbm.at[pl.ds(base, S)], dbuf.at[pl.ds(0, S)])
        lo = jnp.maximum(e0, base) - base
        hi = jnp.minimum(e1, base + S) - base

        def e_body(j, c2):
            d = dbuf[pl.ds(j, L)][0]
            io = lax.broadcasted_iota(jnp.int32, (L,), 0)
            wv = jnp.where(io == 0, wbuf[pl.ds(j, L)], 0.0)
            degbuf[pl.ds(d, L)] = degbuf[pl.ds(d, L)] + wv
            return c2
        lax.fori_loop(lo, hi, e_body, None)
        return carry
    lax.fori_loop(0, nsc, sc_body, None)

    for v in range(NPW // L):
        sl = pl.ds(v * L, L)
        deg = degbuf[sl] + 1.0
        ib = lax.bitcast_convert_type(deg, jnp.int32)
        y = lax.bitcast_convert_type(
            jnp.int32(0x5F3759DF) - lax.shift_right_logical(ib, 1),
            jnp.float32)
        for _ in range(4):
            y = y * (1.5 - 0.5 * deg * y * y)
        dinvbuf[sl] = y
        snbuf[sl] = y * y

    n0 = wid * NPW
    pltpu.sync_copy(dinvbuf, dinv_hbm.at[pl.ds(n0, NPW)])
    pltpu.sync_copy(snbuf, sn_hbm.at[pl.ds(n0, NPW)])


def _deg_kernel(ws, dstl, bounds):
    f = pl.kernel(
        _deg_body,
        out_type=(jax.ShapeDtypeStruct((NPAD,), jnp.float32),
                  jax.ShapeDtypeStruct((NPAD,), jnp.float32)),
        mesh=plsc.VectorSubcoreMesh(**_MESH),
        compiler_params=pltpu.CompilerParams(needs_layout_passes=False),
        scratch_types=[
            pltpu.VMEM((S + L,), jnp.float32),
            pltpu.VMEM((S + L,), jnp.int32),
            pltpu.VMEM((NPW + L,), jnp.float32),
            pltpu.VMEM((48,), jnp.int32),
            pltpu.VMEM((NPW,), jnp.float32),
            pltpu.VMEM((NPW,), jnp.float32),
        ],
    )
    return f(ws, dstl, bounds)


def _norm_body(dinv_hbm, src_hbm, dst_hbm, ws_hbm, norm_hbm,
               tab, sbuf, dbuf, wbuf, nbuf, epw):
    wid = _wid()
    pltpu.sync_copy(dinv_hbm, tab)
    for m in range(epw // S):
        base = wid * epw + m * S
        pltpu.sync_copy(src_hbm.at[pl.ds(base, S)], sbuf)
        pltpu.sync_copy(dst_hbm.at[pl.ds(base, S)], dbuf)
        pltpu.sync_copy(ws_hbm.at[pl.ds(base, S)], wbuf)

        def g_body(u, carry):
            off = pl.ds(u * L, L)
            gs = plsc.load_gather(tab, [sbuf[off]])
            gd = plsc.load_gather(tab, [dbuf[off]])
            nbuf[off] = gs * wbuf[off] * gd
            return carry
        lax.fori_loop(0, S // L, g_body, None)
        pltpu.sync_copy(nbuf, norm_hbm.at[pl.ds(base, S)])


def _norm_kernel(dinv, src_s, dst_s, ws, epad):
    epw = epad // NW
    f = pl.kernel(
        functools.partial(_norm_body, epw=epw),
        out_type=jax.ShapeDtypeStruct((epad,), jnp.float32),
        mesh=plsc.VectorSubcoreMesh(**_MESH),
        compiler_params=pltpu.CompilerParams(needs_layout_passes=False),
        scratch_types=[
            pltpu.VMEM((NPAD,), jnp.float32),
            pltpu.VMEM((S,), jnp.int32),
            pltpu.VMEM((S,), jnp.int32),
            pltpu.VMEM((S,), jnp.float32),
            pltpu.VMEM((S,), jnp.float32),
        ],
    )
    return f(dinv, src_s, dst_s, ws)


def _agg_body(xw_hbm, src_hbm, dstl_hbm, norm_hbm, bounds_hbm, sn_hbm, b_hbm,
              out_hbm, acc, gbuf, sbuf, mdbuf, mnbuf, ibuf, bbuf, snbuf,
              bvbuf, sem0, sem1, relu):
    wid = _wid()
    n0 = wid * NPW
    pltpu.sync_copy(bounds_hbm, bbuf)
    bv = bbuf[pl.ds(wid, L)]
    e0 = bv[0]
    e1 = bv[1]
    pltpu.sync_copy(sn_hbm.at[pl.ds(n0, NPW)], snbuf.at[pl.ds(0, NPW)])
    pltpu.sync_copy(b_hbm, bvbuf)
    cb0 = (e0 // S) * S
    nsc = (e1 - cb0 + S - 1) // S
    sems = [sem0, sem1]

    def t_body(t, carry):
        def zb(i, c2):
            for v in range(VR):
                acc[i, pl.ds(v * L, L)] = jnp.zeros((L,), jnp.float32)
            return c2
        lax.fori_loop(0, NPW, zb, None)

        tN = t * NPAD

        def sc_body(m, c2):
            base = cb0 + m * S
            pltpu.sync_copy(src_hbm.at[pl.ds(base, S)], sbuf)
            pltpu.sync_copy(dstl_hbm.at[pl.ds(base, S)],
                            mdbuf.at[pl.ds(0, S)])
            pltpu.sync_copy(norm_hbm.at[pl.ds(base, S)],
                            mnbuf.at[pl.ds(0, S)])

            def ib_body(u, c3):
                off = pl.ds(u * L, L)
                ibuf[off] = sbuf[off] + tN
                return c3
            lax.fori_loop(0, S // L, ib_body, None)

            lo = jnp.maximum(e0, base)
            hi = jnp.minimum(e1, base + S)

            h = pltpu.async_copy(
                xw_hbm.at[ibuf.at[pl.ds(0, CH)]], gbuf.at[0], sems[0])
            for c in range(NCH):
                p = c % 2
                hn = None
                if c + 1 < NCH:
                    hn = pltpu.async_copy(
                        xw_hbm.at[ibuf.at[pl.ds((c + 1) * CH, CH)]],
                        gbuf.at[1 - p], sems[1 - p])
                h.wait()
                clo = jnp.maximum(lo, base + c * CH) - base
                chi = jnp.minimum(hi, base + (c + 1) * CH) - base

                def e_body(j, c3, p=p, c=c):
                    nb = _bcast(mnbuf[pl.ds(j, L)][0])
                    dd = mdbuf[pl.ds(j, L)][0]
                    jl = j - c * CH
                    for v in range(VR):
                        sl = pl.ds(v * L, L)
                        plsc.addupdate(acc.at[dd, sl], nb * gbuf[p, jl, sl])
                    return c3
                lax.fori_loop(clo, chi, e_body, None)
                h = hn
            return c2
        lax.fori_loop(0, nsc, sc_body, None)

        for q in range(NPW // CH):
            r0 = q * CH
            pltpu.sync_copy(xw_hbm.at[pl.ds(tN + n0 + r0, CH)], gbuf.at[0])

            def s_body(i, c2, r0=r0):
                gi = r0 + i
                sb = _bcast(snbuf[pl.ds(gi, L)][0])
                for v in range(VR):
                    sl = pl.ds(v * L, L)
                    val = acc[gi, sl] + sb * gbuf[0, i, sl] + bvbuf[sl]
                    if relu:
                        val = jnp.maximum(val, 0.0)
                    acc[gi, sl] = val
                return c2
            lax.fori_loop(0, CH, s_body, None)

        pltpu.sync_copy(acc, out_hbm.at[pl.ds(tN + n0, NPW)])
        return carry
    lax.fori_loop(0, T, t_body, None)


def _agg_kernel(xw, src_s, dstl_s, norm_s, bounds, sn, b, relu):
    f = pl.kernel(
        functools.partial(_agg_body, relu=relu),
        out_type=jax.ShapeDtypeStruct((T * NPAD, D), jnp.float32),
        mesh=plsc.VectorSubcoreMesh(**_MESH),
        compiler_params=pltpu.CompilerParams(needs_layout_passes=False),
        scratch_types=[
            pltpu.VMEM((NPW, D), jnp.float32),
            pltpu.VMEM((2, CH, D), jnp.float32),
            pltpu.VMEM((S,), jnp.int32),
            pltpu.VMEM((S + L,), jnp.int32),
            pltpu.VMEM((S + L,), jnp.float32),
            pltpu.VMEM((S,), jnp.int32),
            pltpu.VMEM((48,), jnp.int32),
            pltpu.VMEM((NPW + L,), jnp.float32),
            pltpu.VMEM((D,), jnp.float32),
            pltpu.SemaphoreType.DMA,
            pltpu.SemaphoreType.DMA,
        ],
    )
    return f(xw, src_s, dstl_s, norm_s, bounds, sn, b)


def _mm_body(x_ref, w_ref, o_ref):
    o_ref[...] = jnp.dot(x_ref[...], w_ref[...],
                         preferred_element_type=jnp.float32)


def _matmul(h, W):
    M = h.shape[0]
    return pl.pallas_call(
        _mm_body,
        grid=(M // BM,),
        in_specs=[pl.BlockSpec((BM, D), lambda i: (i, 0)),
                  pl.BlockSpec((D, D), lambda i: (0, 0))],
        out_specs=pl.BlockSpec((BM, D), lambda i: (i, 0)),
        out_shape=jax.ShapeDtypeStruct((M, D), jnp.float32),
    )(h, W)


def kernel(x, mask, spatial_edge_index, spatial_edge_weight, W1, b1, W2, b2):
    src = spatial_edge_index[0].astype(jnp.int32)
    dst = spatial_edge_index[1].astype(jnp.int32)
    w = spatial_edge_weight.astype(jnp.float32)
    E = src.shape[0]

    perm = jnp.argsort(dst)
    src_s = src[perm]
    dst_s = dst[perm]
    w_s = w[perm]
    dstl_s = dst_s % NPW
    bounds = jnp.searchsorted(
        dst_s, jnp.arange(NW + 1, dtype=jnp.int32) * NPW).astype(jnp.int32)
    bounds = jnp.pad(bounds, (0, 48 - (NW + 1)))

    epad = NW * S * -(-E // (NW * S))
    pad = epad - E
    src_p = jnp.pad(src_s, (0, pad))
    dst_p = jnp.pad(dst_s, (0, pad))
    dstl_p = jnp.pad(dstl_s, (0, pad))
    w_p = jnp.pad(w_s, (0, pad))

    dinv, sn = _deg_kernel(w_p, dstl_p, bounds)
    norm_p = _norm_kernel(dinv, src_p, dst_p, w_p, epad)

    h = jnp.pad(x, ((0, 0), (0, NPAD - N_NODES), (0, 0))).reshape(T * NPAD, D)
    xw1 = _matmul(h, W1)
    h1 = _agg_kernel(xw1, src_p, dstl_p, norm_p, bounds, sn, b1, relu=True)
    xw2 = _matmul(h1, W2)
    out = _agg_kernel(xw2, src_p, dstl_p, norm_p, bounds, sn, b2, relu=False)
    return out.reshape(T, NPAD, D)[:, :N_NODES, :]
